# baseline (speedup 1.0000x reference)
import functools

import jax
import jax.numpy as jnp
from jax import lax
from jax.experimental import pallas as pl
from jax.experimental.pallas import tpu as pltpu
from jax.experimental.pallas import tpu_sc as plsc

N_ENT = 100000
N_REL = 100000
H_DIM = 64
BATCH = 16384
REG_PARAM = 0.01

NC = 2
NS = 16
NW = NC * NS
L = 16

PW = BATCH // NW
C = 128
NCHUNK = PW // C
G = C // L
W = 2 * H_DIM
ENT_ROWS = N_ENT // 2


def _sc_body(embp_hbm, wcomb_hbm, hidx_hbm, ridx_hbm, tidx_hbm,
             scores_hbm, sq_hbm,
             hidx_v, ridx_v, tidx_v,
             hrows, trows, crows, scores_v, sq_v, sems):
    wid = lax.axis_index("s") * NC + lax.axis_index("c")
    wbase = wid * PW

    def stage(c, par):
        cbase = wbase + c * C
        pltpu.sync_copy(hidx_hbm.at[pl.ds(cbase, C)], hidx_v.at[par])
        pltpu.sync_copy(ridx_hbm.at[pl.ds(cbase, C)], ridx_v.at[par])
        pltpu.sync_copy(tidx_hbm.at[pl.ds(cbase, C)], tidx_v.at[par])
        sem = sems.at[par]
        return (
            pltpu.async_copy(embp_hbm.at[hidx_v.at[par]], hrows.at[par], sem),
            pltpu.async_copy(embp_hbm.at[tidx_v.at[par]], trows.at[par], sem),
            pltpu.async_copy(wcomb_hbm.at[ridx_v.at[par]], crows.at[par], sem),
        )

    def compute(c, par, sq_acc):
        def group(g, sq):
            rows = g * L + lax.iota(jnp.int32, L)
            score = jnp.zeros((L,), jnp.float32)
            for j in range(H_DIM):
                cj = jnp.full((L,), j, jnp.int32)
                vh = plsc.load_gather(hrows.at[par], [rows, cj])
                vt = plsc.load_gather(trows.at[par], [rows, cj])
                vr = plsc.load_gather(crows.at[par], [rows, cj])
                vi = plsc.load_gather(crows.at[par], [rows, cj + H_DIM])
                score = score + vh * vt * (vr + vi)
                sq = sq + (vh * vh + vt * vt) + (vr * vr + vi * vi)
            scores_v[pl.ds(c * C + g * L, L)] = score * 0.5
            return sq

        return lax.fori_loop(0, G, group, sq_acc)

    sq_acc = jnp.zeros((L,), jnp.float32)
    handles = stage(0, 0)
    for c in range(NCHUNK):
        par = c & 1
        next_handles = stage(c + 1, 1 - par) if c + 1 < NCHUNK else None
        for h in handles:
            h.wait()
        sq_acc = compute(c, par, sq_acc)
        handles = next_handles

    sq_v[...] = sq_acc
    pltpu.sync_copy(scores_v, scores_hbm.at[pl.ds(wbase, PW)])
    pltpu.sync_copy(sq_v, sq_hbm.at[pl.ds(wid * L, L)])


@jax.jit
def _sc_gather_score(embp, wcomb, hidx, ridx, tidx):
    mesh = plsc.VectorSubcoreMesh(core_axis_name="c", subcore_axis_name="s")
    f = functools.partial(
        pl.kernel,
        out_type=[
            jax.ShapeDtypeStruct((BATCH,), jnp.float32),
            jax.ShapeDtypeStruct((NW * L,), jnp.float32),
        ],
        mesh=mesh,
        compiler_params=pltpu.CompilerParams(needs_layout_passes=False),
        scratch_types=[
            pltpu.VMEM((2, C), jnp.int32),
            pltpu.VMEM((2, C), jnp.int32),
            pltpu.VMEM((2, C), jnp.int32),
            pltpu.VMEM((2, C, W), jnp.float32),
            pltpu.VMEM((2, C, W), jnp.float32),
            pltpu.VMEM((2, C, W), jnp.float32),
            pltpu.VMEM((PW,), jnp.float32),
            pltpu.VMEM((L,), jnp.float32),
            pltpu.SemaphoreType.DMA((2,)),
        ],
    )(_sc_body)
    return f(embp, wcomb, hidx, ridx, tidx)


def _tc_loss_body(s_ref, l_ref, q_ref, o_ref):
    s = s_ref[...]
    lbl = l_ref[...]
    bce = jnp.maximum(s, 0.0) - s * lbl + jnp.log1p(jnp.exp(-jnp.abs(s)))
    predict_loss = jnp.sum(bce) / BATCH
    reg_loss = jnp.sum(q_ref[...]) / (4.0 * BATCH * H_DIM)
    o_ref[0, 0] = predict_loss + REG_PARAM * reg_loss


@jax.jit
def _tc_loss(scores, labels, sqsums):
    out = pl.pallas_call(
        _tc_loss_body,
        out_shape=jax.ShapeDtypeStruct((1, 1), jnp.float32),
        out_specs=pl.BlockSpec(memory_space=pltpu.SMEM),
    )(scores.reshape(128, 128), labels.reshape(128, 128),
      sqsums.reshape(4, 128))
    return out[0, 0]


def kernel(mixedEmbedding, w_relation, w_relation_inv, triplets, labels):
    embp = jnp.concatenate([mixedEmbedding, mixedEmbedding], axis=1)
    wcomb = jnp.concatenate([w_relation, w_relation_inv], axis=1)
    hidx = triplets[:, 0]
    ridx = triplets[:, 1]
    tidx = triplets[:, 2]
    scores, sqsums = _sc_gather_score(embp, wcomb, hidx, ridx, tidx)
    return _tc_loss(scores, labels, sqsums)


# speedup vs baseline: 1.1341x; 1.1341x over previous
import functools

import jax
import jax.numpy as jnp
from jax import lax
from jax.experimental import pallas as pl
from jax.experimental.pallas import tpu as pltpu
from jax.experimental.pallas import tpu_sc as plsc

N_ENT = 100000
N_REL = 100000
H_DIM = 64
BATCH = 16384
REG_PARAM = 0.01

NC = 2
NS = 16
NW = NC * NS
L = 16

PW = BATCH // NW
C = 128
NCHUNK = PW // C
G = C // L
W = 2 * H_DIM


def _sc_body(embp_hbm, wcomb_hbm, hidx_hbm, ridx_hbm, tidx_hbm,
             scores_hbm, sq_hbm,
             hidx_v, ridx_v, tidx_v, hhalf_v, thalf_v, hcol_v, tcol_v,
             hrows, trows, crows, scores_v, sq_v, sems):
    wid = lax.axis_index("s") * NC + lax.axis_index("c")
    wbase = wid * PW

    def stage(c, par):
        cbase = wbase + c * C
        pltpu.sync_copy(hidx_hbm.at[pl.ds(cbase, C)], hidx_v.at[par])
        pltpu.sync_copy(ridx_hbm.at[pl.ds(cbase, C)], ridx_v.at[par])
        pltpu.sync_copy(tidx_hbm.at[pl.ds(cbase, C)], tidx_v.at[par])
        for g in range(G):
            sl = pl.ds(g * L, L)
            h = hidx_v.at[par][sl]
            t = tidx_v.at[par][sl]
            hhalf_v.at[par][sl] = lax.shift_right_logical(h, 1)
            thalf_v.at[par][sl] = lax.shift_right_logical(t, 1)
            hcol_v.at[par][sl] = (h & 1) * H_DIM
            tcol_v.at[par][sl] = (t & 1) * H_DIM
        sem = sems.at[par]
        return (
            pltpu.async_copy(embp_hbm.at[hhalf_v.at[par]], hrows.at[par], sem),
            pltpu.async_copy(embp_hbm.at[thalf_v.at[par]], trows.at[par], sem),
            pltpu.async_copy(wcomb_hbm.at[ridx_v.at[par]], crows.at[par], sem),
        )

    def compute(c, par, sq_acc):
        def group(g, sq):
            rows = g * L + lax.iota(jnp.int32, L)
            hcol = plsc.load_gather(hcol_v.at[par], [rows])
            tcol = plsc.load_gather(tcol_v.at[par], [rows])
            score = jnp.zeros((L,), jnp.float32)
            for j in range(H_DIM):
                cj = jnp.full((L,), j, jnp.int32)
                vh = plsc.load_gather(hrows.at[par], [rows, hcol + j])
                vt = plsc.load_gather(trows.at[par], [rows, tcol + j])
                vr = plsc.load_gather(crows.at[par], [rows, cj])
                vi = plsc.load_gather(crows.at[par], [rows, cj + H_DIM])
                score = score + vh * vt * (vr + vi)
                sq = sq + (vh * vh + vt * vt) + (vr * vr + vi * vi)
            scores_v[pl.ds(c * C + g * L, L)] = score * 0.5
            return sq

        return lax.fori_loop(0, G, group, sq_acc)

    sq_acc = jnp.zeros((L,), jnp.float32)
    handles = stage(0, 0)
    for c in range(NCHUNK):
        par = c & 1
        next_handles = stage(c + 1, 1 - par) if c + 1 < NCHUNK else None
        for h in handles:
            h.wait()
        sq_acc = compute(c, par, sq_acc)
        handles = next_handles

    sq_v[...] = sq_acc
    pltpu.sync_copy(scores_v, scores_hbm.at[pl.ds(wbase, PW)])
    pltpu.sync_copy(sq_v, sq_hbm.at[pl.ds(wid * L, L)])


@jax.jit
def _sc_gather_score(embp, wcomb, hidx, ridx, tidx):
    mesh = plsc.VectorSubcoreMesh(core_axis_name="c", subcore_axis_name="s")
    f = functools.partial(
        pl.kernel,
        out_type=[
            jax.ShapeDtypeStruct((BATCH,), jnp.float32),
            jax.ShapeDtypeStruct((NW * L,), jnp.float32),
        ],
        mesh=mesh,
        compiler_params=pltpu.CompilerParams(needs_layout_passes=False),
        scratch_types=[
            pltpu.VMEM((2, C), jnp.int32),
            pltpu.VMEM((2, C), jnp.int32),
            pltpu.VMEM((2, C), jnp.int32),
            pltpu.VMEM((2, C), jnp.int32),
            pltpu.VMEM((2, C), jnp.int32),
            pltpu.VMEM((2, C), jnp.int32),
            pltpu.VMEM((2, C), jnp.int32),
            pltpu.VMEM((2, C, W), jnp.float32),
            pltpu.VMEM((2, C, W), jnp.float32),
            pltpu.VMEM((2, C, W), jnp.float32),
            pltpu.VMEM((PW,), jnp.float32),
            pltpu.VMEM((L,), jnp.float32),
            pltpu.SemaphoreType.DMA((2,)),
        ],
    )(_sc_body)
    return f(embp, wcomb, hidx, ridx, tidx)


def _tc_loss_body(s_ref, l_ref, q_ref, o_ref):
    s = s_ref[...]
    lbl = l_ref[...]
    bce = jnp.maximum(s, 0.0) - s * lbl + jnp.log1p(jnp.exp(-jnp.abs(s)))
    predict_loss = jnp.sum(bce) / BATCH
    reg_loss = jnp.sum(q_ref[...]) / (4.0 * BATCH * H_DIM)
    o_ref[0, 0] = predict_loss + REG_PARAM * reg_loss


@jax.jit
def _tc_loss(scores, labels, sqsums):
    out = pl.pallas_call(
        _tc_loss_body,
        out_shape=jax.ShapeDtypeStruct((1, 1), jnp.float32),
        out_specs=pl.BlockSpec(memory_space=pltpu.SMEM),
    )(scores.reshape(128, 128), labels.reshape(128, 128),
      sqsums.reshape(4, 128))
    return out[0, 0]


def kernel(mixedEmbedding, w_relation, w_relation_inv, triplets, labels):
    embp = mixedEmbedding.reshape(N_ENT // 2, W)
    wcomb = jnp.concatenate([w_relation, w_relation_inv], axis=1)
    hidx = triplets[:, 0]
    ridx = triplets[:, 1]
    tidx = triplets[:, 2]
    scores, sqsums = _sc_gather_score(embp, wcomb, hidx, ridx, tidx)
    return _tc_loss(scores, labels, sqsums)
